# Initial kernel scaffold; baseline (speedup 1.0000x reference)
import jax
import jax.numpy as jnp
from jax.experimental import pallas as pl


def kernel(madis_x, madis_lon, madis_lat, edge_index, ex_lon, ex_lat, ex_x, edge_index_e2m, params):
    raise NotImplementedError("write your pallas kernel here")


# speedup vs baseline: 1.3886x; 1.3886x over previous
import functools

import jax
import jax.numpy as jnp
from jax import lax
from jax.experimental import pallas as pl
from jax.experimental.pallas import tpu as pltpu
from jax.experimental.pallas import tpu_sc as plsc

HID = 128
NC, NS, LANES = 2, 16, 16
NW = NC * NS
KB = 128
BN = 1024
FS = HID // 4


def _mm(a, b):
    return jnp.dot(a, b, preferred_element_type=jnp.float32)


def _bf(z):
    return z.astype(jnp.bfloat16).astype(jnp.float32)


def _pos_term(pos, wp):
    return (_bf(pos[:, 0:1]) * _bf(wp[0:1, :])
            + _bf(pos[:, 1:2]) * _bf(wp[1:2, :]))


def _row_call(body, row_ins, full_ins, out_dims):
    nrows = row_ins[0].shape[0]
    grid = (nrows // BN,)
    in_specs = ([pl.BlockSpec((BN, a.shape[1]), lambda i: (i, 0)) for a in row_ins]
                + [pl.BlockSpec(a.shape, lambda i: (0, 0)) for a in full_ins])
    out_shape = [jax.ShapeDtypeStruct((nrows, d), jnp.float32) for d in out_dims]
    out_specs = [pl.BlockSpec((BN, d), lambda i: (i, 0)) for d in out_dims]
    outs = pl.pallas_call(body, grid=grid, in_specs=in_specs,
                          out_specs=out_specs, out_shape=out_shape)(
        *row_ins, *full_ins)
    return outs


def _emb_body(u, pos, wu, wp, b1, w2, b2, o):
    pp = _pos_term(pos, wp)
    h = jnp.tanh(_mm(u[...], wu[...]) + pp + b1[...])
    o[...] = jnp.tanh(_mm(h, w2[...]) + b2[...])


def _int_proj_body(x, ws, wd, b1, s, d):
    s[...] = _mm(x[...], ws[...])
    d[...] = _mm(x[...], wd[...]) + b1[...]


def _ext_dproj_body(x, wx, b1, d):
    d[...] = _mm(x[...], wx[...]) + b1[...]


def _ext_sproj_body(exf, we, s):
    s[...] = _mm(exf[...], we[...])


def _edge_body(pre, w2, b2, o):
    h = jnp.tanh(pre[...])
    o[...] = jnp.tanh(_mm(h, w2[...]) + b2[...])


def _upd_int_body(x, agg, u, ux, ua, uu, c1, u2, c2, o):
    h = jnp.tanh(_mm(x[...], ux[...]) + _mm(agg[...], ua[...])
                 + _mm(u[...], uu[...]) + c1[...])
    o[...] = x[...] + _mm(h, u2[...]) + c2[...]


def _upd_ext_body(x, agg, ux, ua, c1, u2, c2, o):
    h = jnp.tanh(_mm(x[...], ux[...]) + _mm(agg[...], ua[...]) + c1[...])
    o[...] = x[...] + _mm(h, u2[...]) + c2[...]


def _out_body(x, w1, b1, w2, b2, o):
    h = jnp.tanh(_mm(x[...], w1[...]) + b1[...])
    o[...] = _mm(h, w2[...]) + b2[...]


def _sc_mesh():
    return plsc.VectorSubcoreMesh(core_axis_name="c", subcore_axis_name="s",
                                  num_cores=NC, num_subcores=NS)


def _sc_bf(z):
    return lax.convert_element_type(
        lax.convert_element_type(z, jnp.bfloat16), jnp.float32)


def _gather_add(sproj, dproj, pa, pb, wp2, src, dst):
    e_pad = src.shape[0]
    ew = e_pad // NW
    nb = ew // KB

    @functools.partial(
        pl.kernel, mesh=_sc_mesh(),
        out_type=jax.ShapeDtypeStruct((e_pad, HID), jnp.float32),
        compiler_params=pltpu.CompilerParams(use_tc_tiling_on_sc=False),
        scratch_types=[
            pltpu.VMEM((KB,), jnp.int32),
            pltpu.VMEM((KB,), jnp.int32),
            pltpu.VMEM((KB, HID), jnp.float32),
            pltpu.VMEM((KB, HID), jnp.float32),
            pltpu.VMEM((KB, 16), jnp.float32),
            pltpu.VMEM((KB, 16), jnp.float32),
            pltpu.VMEM((2, HID), jnp.float32),
            pltpu.SemaphoreType.DMA,
            pltpu.SemaphoreType.DMA,
            pltpu.SemaphoreType.DMA,
            pltpu.SemaphoreType.DMA,
        ])
    def k(sproj_h, dproj_h, pa_h, pb_h, wp_h, src_h, dst_h, pre_h,
          si, di, av, bv, pav, pbv, wpv, sem_a, sem_b, sem_c, sem_d):
        wid = lax.axis_index("s") * NC + lax.axis_index("c")
        base = pl.multiple_of(wid * ew, KB)
        pltpu.sync_copy(wp_h, wpv)

        def body(i, _):
            off = pl.multiple_of(base + i * KB, KB)
            pltpu.sync_copy(src_h.at[pl.ds(off, KB)], si)
            pltpu.sync_copy(dst_h.at[pl.ds(off, KB)], di)
            cpa = pltpu.async_copy(sproj_h.at[si], av, sem_a)
            cpb = pltpu.async_copy(dproj_h.at[di], bv, sem_b)
            cpc = pltpu.async_copy(pa_h.at[si], pav, sem_c)
            cpd = pltpu.async_copy(pb_h.at[di], pbv, sem_d)
            cpa.wait()
            cpb.wait()
            cpc.wait()
            cpd.wait()

            def row(r, _):
                dd = _sc_bf(pav[r] - pbv[r])
                d0 = dd[0]
                d1 = dd[1]
                for j in range(HID // LANES):
                    sl = pl.ds(j * LANES, LANES)
                    av[r, sl] = (av[r, sl] + bv[r, sl]
                                 + d0 * wpv[0, sl] + d1 * wpv[1, sl])
                return 0

            lax.fori_loop(0, KB, row, 0)
            pltpu.sync_copy(av, pre_h.at[pl.ds(off, KB)])
            return 0

        lax.fori_loop(0, nb, body, 0)

    return k(sproj, dproj, pa, pb, wp2, src, dst)


def _segment_sum(msg, dst, n_pad):
    e_pad = dst.shape[0]
    ew = e_pad // NS
    nb = ew // KB
    zr = n_pad // NS // 8
    wr = n_pad // NS

    @functools.partial(
        pl.kernel, mesh=_sc_mesh(),
        out_type=jax.ShapeDtypeStruct((n_pad, HID), jnp.float32),
        compiler_params=pltpu.CompilerParams(use_tc_tiling_on_sc=False),
        scratch_types=[
            pltpu.VMEM((KB,), jnp.int32),
            pltpu.VMEM((KB, FS), jnp.float32),
            pltpu.VMEM((zr, FS), jnp.float32),
            pltpu.VMEM_SHARED((n_pad, FS), jnp.float32),
        ])
    def k(msg_h, dst_h, agg_h, iv, rows, zb, acc):
        cid = lax.axis_index("c")
        sid = lax.axis_index("s")

        def zrow(r, _):
            for j in range(FS // LANES):
                zb[r, pl.ds(j * LANES, LANES)] = jnp.zeros((LANES,), jnp.float32)
            return 0

        lax.fori_loop(0, zr, zrow, 0)

        for p in range(2):
            col = (cid * 2 + p) * FS
            for zc in range(8):
                pltpu.sync_copy(zb, acc.at[pl.ds(sid * wr + zc * zr, zr)])
            plsc.subcore_barrier()
            ebase = pl.multiple_of(sid * ew, KB)

            def body(i, _):
                off = pl.multiple_of(ebase + i * KB, KB)
                pltpu.sync_copy(dst_h.at[pl.ds(off, KB)], iv)
                pltpu.sync_copy(msg_h.at[pl.ds(off, KB), pl.ds(col, FS)], rows)
                pltpu.sync_copy(rows, acc.at[iv], add=True)
                return 0

            lax.fori_loop(0, nb, body, 0)
            plsc.subcore_barrier()
            pltpu.sync_copy(acc.at[pl.ds(sid * wr, wr)],
                            agg_h.at[pl.ds(sid * wr, wr), pl.ds(col, FS)])
            plsc.subcore_barrier()

    return k(msg, dst)


def _pad_rows(a, n):
    return jnp.pad(a, ((0, n - a.shape[0]), (0, 0)))


def _pad_edges(src, dst, e_pad, pad_node):
    e = src.shape[0]
    src = jnp.pad(src, (0, e_pad - e))
    dst = jnp.pad(dst, (0, e_pad - e), constant_values=pad_node)
    return src, dst


def _round_up(x, m):
    return (x + m - 1) // m * m


def kernel(madis_x, madis_lon, madis_lat, edge_index, ex_lon, ex_lat, ex_x,
           edge_index_e2m, params):
    B, Nm = madis_x.shape[0], madis_x.shape[1]
    Ne = ex_x.shape[1]
    N = B * Nm
    NE = B * Ne
    n_pad = _round_up(N, BN)
    ne_pad = _round_up(NE, BN)

    u = _pad_rows(madis_x.reshape(N, -1), n_pad)
    pos = _pad_rows(jnp.concatenate([madis_lon, madis_lat], axis=2).reshape(N, 2),
                    n_pad)
    exf = _pad_rows(ex_x.reshape(NE, -1), ne_pad)
    ex_pos = _pad_rows(
        jnp.concatenate([ex_lon[..., None], ex_lat[..., None]], axis=2)
        .reshape(NE, 2), ne_pad)

    sh_m = (jnp.arange(B, dtype=jnp.int32) * Nm)[:, None]
    src_m = (edge_index[:, 0, :] + sh_m).reshape(-1)
    dst_m = (edge_index[:, 1, :] + sh_m).reshape(-1)
    sh_e = (jnp.arange(B, dtype=jnp.int32) * Ne)[:, None]
    src_e = (edge_index_e2m[:, 0, :] + sh_e).reshape(-1)
    dst_e = (edge_index_e2m[:, 1, :] + sh_m).reshape(-1)
    em_pad = _round_up(src_m.shape[0], NW * KB)
    ee_pad = _round_up(src_e.shape[0], NW * KB)
    src_m, dst_m = _pad_edges(src_m, dst_m, em_pad, N)
    src_e, dst_e = _pad_edges(src_e, dst_e, ee_pad, N)

    pos16 = jnp.pad(pos, ((0, 0), (0, 14)))
    ex_pos16 = jnp.pad(ex_pos, ((0, 0), (0, 14)))

    p = params

    def w(t):
        W, b = t
        return W, b.reshape(1, -1)

    (W1, b1), (W2, b2) = map(w, p['emb'])
    x, = _row_call(_emb_body, [u, pos],
                   [W1[:-2], W1[-2:], b1, W2, b2], [HID])

    def ext_layer(lp, x):
        (W1, b1), (W2, b2) = map(w, lp['msg'])
        (U1, c1), (U2, c2) = map(w, lp['upd'])
        dproj, = _row_call(_ext_dproj_body, [x], [W1[:HID], b1], [HID])
        sproj, = _row_call(_ext_sproj_body, [exf], [W1[HID:HID + 8]], [HID])
        wp2 = -_bf(W1[HID + 8:])
        pre = _gather_add(sproj, dproj, ex_pos16, pos16, wp2, src_e, dst_e)
        msg, = _row_call(_edge_body, [pre], [W2, b2], [HID])
        agg = _segment_sum(msg, dst_e, n_pad)
        xn, = _row_call(_upd_ext_body, [x, agg],
                        [U1[:HID], U1[HID:], c1, U2, c2], [HID])
        return xn

    def int_layer(lp, x):
        (W1, b1), (W2, b2) = map(w, lp['msg'])
        (U1, c1), (U2, c2) = map(w, lp['upd'])
        sproj, dproj = _row_call(_int_proj_body, [x],
                                 [W1[:HID], W1[HID:2 * HID], b1], [HID, HID])
        wp2 = _bf(W1[2 * HID:])
        pre = _gather_add(sproj, dproj, pos16, pos16, wp2, src_m, dst_m)
        msg, = _row_call(_edge_body, [pre], [W2, b2], [HID])
        agg = _segment_sum(msg, dst_m, n_pad)
        xn, = _row_call(_upd_int_body, [x, agg, u],
                        [U1[:HID], U1[HID:2 * HID], U1[2 * HID:], c1, U2, c2],
                        [HID])
        return xn

    x = ext_layer(p['ex1'], x)
    for lp in p['internal']:
        x = int_layer(lp, x)
    x = ext_layer(p['ex2'], x)

    (O1, o1), (O2, o2) = map(w, p['out'])
    out, = _row_call(_out_body, [x], [O1, o1, O2, o2], [5])
    return out[:N].reshape(B, Nm, 5)


# speedup vs baseline: 1.9281x; 1.3886x over previous
import functools

import jax
import jax.numpy as jnp
from jax import lax
from jax.experimental import pallas as pl
from jax.experimental.pallas import tpu as pltpu
from jax.experimental.pallas import tpu_sc as plsc

HID = 128
NC, NS, LANES = 2, 16, 16
NW = NC * NS
KB = 128
BN = 1024
FS = HID // 4


def _mm(a, b):
    return jnp.dot(a, b, preferred_element_type=jnp.float32)


def _bf(z):
    return z.astype(jnp.bfloat16).astype(jnp.float32)


def _pos_term(pos, wp):
    return (_bf(pos[:, 0:1]) * _bf(wp[0:1, :])
            + _bf(pos[:, 1:2]) * _bf(wp[1:2, :]))


def _row_call(body, row_ins, full_ins, out_dims):
    nrows = row_ins[0].shape[0]
    grid = (nrows // BN,)
    in_specs = ([pl.BlockSpec((BN, a.shape[1]), lambda i: (i, 0)) for a in row_ins]
                + [pl.BlockSpec(a.shape, lambda i: (0, 0)) for a in full_ins])
    out_shape = [jax.ShapeDtypeStruct((nrows, d), jnp.float32) for d in out_dims]
    out_specs = [pl.BlockSpec((BN, d), lambda i: (i, 0)) for d in out_dims]
    outs = pl.pallas_call(body, grid=grid, in_specs=in_specs,
                          out_specs=out_specs, out_shape=out_shape)(
        *row_ins, *full_ins)
    return outs


def _emb_body(u, pos, wu, wp, b1, w2, b2, o):
    pp = _pos_term(pos, wp)
    h = jnp.tanh(_mm(u[...], wu[...]) + pp + b1[...])
    o[...] = jnp.tanh(_mm(h, w2[...]) + b2[...])


def _int_proj_body(x, ws, wd, b1, s, d):
    s[...] = _mm(x[...], ws[...])
    d[...] = _mm(x[...], wd[...]) + b1[...]


def _ext_dproj_body(x, wx, b1, d):
    d[...] = _mm(x[...], wx[...]) + b1[...]


def _ext_sproj_body(exf, we, s):
    s[...] = _mm(exf[...], we[...])


def _edge_body(pre, w2, b2, o):
    h = jnp.tanh(pre[...])
    o[...] = jnp.tanh(_mm(h, w2[...]) + b2[...])


def _upd_int_body(x, agg, u, ux, ua, uu, c1, u2, c2, o):
    h = jnp.tanh(_mm(x[...], ux[...]) + _mm(agg[...], ua[...])
                 + _mm(u[...], uu[...]) + c1[...])
    o[...] = x[...] + _mm(h, u2[...]) + c2[...]


def _upd_ext_body(x, agg, ux, ua, c1, u2, c2, o):
    h = jnp.tanh(_mm(x[...], ux[...]) + _mm(agg[...], ua[...]) + c1[...])
    o[...] = x[...] + _mm(h, u2[...]) + c2[...]


def _out_body(x, w1, b1, w2, b2, o):
    h = jnp.tanh(_mm(x[...], w1[...]) + b1[...])
    o[...] = _mm(h, w2[...]) + b2[...]


def _sc_mesh():
    return plsc.VectorSubcoreMesh(core_axis_name="c", subcore_axis_name="s",
                                  num_cores=NC, num_subcores=NS)


def _sc_bf(z):
    return lax.convert_element_type(
        lax.convert_element_type(z, jnp.bfloat16), jnp.float32)


def _gather_add(sproj, dproj, pa, pb, wp2, src, dst):
    e_pad = src.shape[0]
    ew = e_pad // NW
    nb = ew // KB

    @functools.partial(
        pl.kernel, mesh=_sc_mesh(),
        out_type=jax.ShapeDtypeStruct((e_pad, HID), jnp.float32),
        compiler_params=pltpu.CompilerParams(use_tc_tiling_on_sc=False),
        scratch_types=[
            [pltpu.VMEM((KB,), jnp.int32)] * 2,
            [pltpu.VMEM((KB,), jnp.int32)] * 2,
            [pltpu.VMEM((KB, HID), jnp.float32)] * 2,
            [pltpu.VMEM((KB, HID), jnp.float32)] * 2,
            [pltpu.VMEM((KB, 16), jnp.float32)] * 2,
            [pltpu.VMEM((KB, 16), jnp.float32)] * 2,
            pltpu.VMEM((2, HID), jnp.float32),
            [pltpu.SemaphoreType.DMA] * 2,
            [pltpu.SemaphoreType.DMA] * 2,
            [pltpu.SemaphoreType.DMA] * 2,
            [pltpu.SemaphoreType.DMA] * 2,
        ])
    def k(sproj_h, dproj_h, pa_h, pb_h, wp_h, src_h, dst_h, pre_h,
          si, di, av, bv, pav, pbv, wpv, sem_a, sem_b, sem_c, sem_d):
        wid = lax.axis_index("s") * NC + lax.axis_index("c")
        base = pl.multiple_of(wid * ew, KB)
        pltpu.sync_copy(wp_h, wpv)

        def issue(i, s):
            off = pl.multiple_of(base + i * KB, KB)
            pltpu.sync_copy(src_h.at[pl.ds(off, KB)], si[s])
            pltpu.sync_copy(dst_h.at[pl.ds(off, KB)], di[s])
            pltpu.async_copy(sproj_h.at[si[s]], av[s], sem_a[s])
            pltpu.async_copy(dproj_h.at[di[s]], bv[s], sem_b[s])
            pltpu.async_copy(pa_h.at[si[s]], pav[s], sem_c[s])
            pltpu.async_copy(pb_h.at[di[s]], pbv[s], sem_d[s])

        issue(0, 0)

        def half(i, s):
            pltpu.make_async_copy(sproj_h.at[si[s]], av[s], sem_a[s]).wait()
            pltpu.make_async_copy(dproj_h.at[di[s]], bv[s], sem_b[s]).wait()
            pltpu.make_async_copy(pa_h.at[si[s]], pav[s], sem_c[s]).wait()
            pltpu.make_async_copy(pb_h.at[di[s]], pbv[s], sem_d[s]).wait()

            def row(r, _):
                dd = _sc_bf(pav[s][r] - pbv[s][r])
                d0 = dd[0]
                d1 = dd[1]
                for j in range(HID // LANES):
                    sl = pl.ds(j * LANES, LANES)
                    av[s][r, sl] = (av[s][r, sl] + bv[s][r, sl]
                                    + d0 * wpv[0, sl] + d1 * wpv[1, sl])
                return 0

            lax.fori_loop(0, KB, row, 0)
            off = pl.multiple_of(base + i * KB, KB)
            pltpu.sync_copy(av[s], pre_h.at[pl.ds(off, KB)])

        def body(g, _):
            i0 = g * 2
            issue(i0 + 1, 1)
            half(i0, 0)

            @pl.when(i0 + 2 < nb)
            def _():
                issue(i0 + 2, 0)

            half(i0 + 1, 1)
            return 0

        lax.fori_loop(0, nb // 2, body, 0)

    return k(sproj, dproj, pa, pb, wp2, src, dst)


def _segment_sum(msg, dst, n_pad):
    e_pad = dst.shape[0]
    ew = e_pad // NS
    nb = ew // KB
    zr = n_pad // NS // 8
    wr = n_pad // NS

    @functools.partial(
        pl.kernel, mesh=_sc_mesh(),
        out_type=jax.ShapeDtypeStruct((n_pad, HID), jnp.float32),
        compiler_params=pltpu.CompilerParams(use_tc_tiling_on_sc=False),
        scratch_types=[
            [pltpu.VMEM((KB,), jnp.int32)] * 2,
            [pltpu.VMEM((KB, FS), jnp.float32)] * 2,
            pltpu.VMEM((zr, FS), jnp.float32),
            pltpu.VMEM_SHARED((n_pad, FS), jnp.float32),
            [pltpu.SemaphoreType.DMA] * 2,
            [pltpu.SemaphoreType.DMA] * 2,
        ])
    def k(msg_h, dst_h, agg_h, iv, rows, zb, acc, sem_i, sem_r):
        cid = lax.axis_index("c")
        sid = lax.axis_index("s")

        def zrow(r, _):
            for j in range(FS // LANES):
                zb[r, pl.ds(j * LANES, LANES)] = jnp.zeros((LANES,), jnp.float32)
            return 0

        lax.fori_loop(0, zr, zrow, 0)
        ebase = pl.multiple_of(sid * ew, KB)

        for p in range(2):
            col = (cid * 2 + p) * FS
            for zc in range(8):
                pltpu.sync_copy(zb, acc.at[pl.ds(sid * wr + zc * zr, zr)])
            plsc.subcore_barrier()

            def issue(i, s):
                off = pl.multiple_of(ebase + i * KB, KB)
                pltpu.async_copy(dst_h.at[pl.ds(off, KB)], iv[s], sem_i[s])
                pltpu.async_copy(msg_h.at[pl.ds(off, KB), pl.ds(col, FS)],
                                 rows[s], sem_r[s])

            issue(0, 0)

            def half(i, s):
                off = pl.multiple_of(ebase + i * KB, KB)
                pltpu.make_async_copy(dst_h.at[pl.ds(off, KB)], iv[s],
                                      sem_i[s]).wait()
                pltpu.make_async_copy(
                    msg_h.at[pl.ds(off, KB), pl.ds(col, FS)], rows[s],
                    sem_r[s]).wait()
                pltpu.sync_copy(rows[s], acc.at[iv[s]], add=True)

            def body(g, _):
                i0 = g * 2
                issue(i0 + 1, 1)
                half(i0, 0)

                @pl.when(i0 + 2 < nb)
                def _():
                    issue(i0 + 2, 0)

                half(i0 + 1, 1)
                return 0

            lax.fori_loop(0, nb // 2, body, 0)
            plsc.subcore_barrier()
            pltpu.sync_copy(acc.at[pl.ds(sid * wr, wr)],
                            agg_h.at[pl.ds(sid * wr, wr), pl.ds(col, FS)])
            plsc.subcore_barrier()

    return k(msg, dst)


def _pad_rows(a, n):
    return jnp.pad(a, ((0, n - a.shape[0]), (0, 0)))


def _pad_edges(src, dst, e_pad, pad_node):
    e = src.shape[0]
    src = jnp.pad(src, (0, e_pad - e))
    dst = jnp.pad(dst, (0, e_pad - e), constant_values=pad_node)
    return src, dst


def _round_up(x, m):
    return (x + m - 1) // m * m


def kernel(madis_x, madis_lon, madis_lat, edge_index, ex_lon, ex_lat, ex_x,
           edge_index_e2m, params):
    B, Nm = madis_x.shape[0], madis_x.shape[1]
    Ne = ex_x.shape[1]
    N = B * Nm
    NE = B * Ne
    n_pad = _round_up(N, BN)
    ne_pad = _round_up(NE, BN)

    u = _pad_rows(madis_x.reshape(N, -1), n_pad)
    pos = _pad_rows(jnp.concatenate([madis_lon, madis_lat], axis=2).reshape(N, 2),
                    n_pad)
    exf = _pad_rows(ex_x.reshape(NE, -1), ne_pad)
    ex_pos = _pad_rows(
        jnp.concatenate([ex_lon[..., None], ex_lat[..., None]], axis=2)
        .reshape(NE, 2), ne_pad)

    sh_m = (jnp.arange(B, dtype=jnp.int32) * Nm)[:, None]
    src_m = (edge_index[:, 0, :] + sh_m).reshape(-1)
    dst_m = (edge_index[:, 1, :] + sh_m).reshape(-1)
    sh_e = (jnp.arange(B, dtype=jnp.int32) * Ne)[:, None]
    src_e = (edge_index_e2m[:, 0, :] + sh_e).reshape(-1)
    dst_e = (edge_index_e2m[:, 1, :] + sh_m).reshape(-1)
    em_pad = _round_up(src_m.shape[0], 2 * NW * KB)
    ee_pad = _round_up(src_e.shape[0], 2 * NW * KB)
    src_m, dst_m = _pad_edges(src_m, dst_m, em_pad, N)
    src_e, dst_e = _pad_edges(src_e, dst_e, ee_pad, N)

    pos16 = jnp.pad(pos, ((0, 0), (0, 14)))
    ex_pos16 = jnp.pad(ex_pos, ((0, 0), (0, 14)))

    p = params

    def w(t):
        W, b = t
        return W, b.reshape(1, -1)

    (W1, b1), (W2, b2) = map(w, p['emb'])
    x, = _row_call(_emb_body, [u, pos],
                   [W1[:-2], W1[-2:], b1, W2, b2], [HID])

    def ext_layer(lp, x):
        (W1, b1), (W2, b2) = map(w, lp['msg'])
        (U1, c1), (U2, c2) = map(w, lp['upd'])
        dproj, = _row_call(_ext_dproj_body, [x], [W1[:HID], b1], [HID])
        sproj, = _row_call(_ext_sproj_body, [exf], [W1[HID:HID + 8]], [HID])
        wp2 = -_bf(W1[HID + 8:])
        pre = _gather_add(sproj, dproj, ex_pos16, pos16, wp2, src_e, dst_e)
        msg, = _row_call(_edge_body, [pre], [W2, b2], [HID])
        agg = _segment_sum(msg, dst_e, n_pad)
        xn, = _row_call(_upd_ext_body, [x, agg],
                        [U1[:HID], U1[HID:], c1, U2, c2], [HID])
        return xn

    def int_layer(lp, x):
        (W1, b1), (W2, b2) = map(w, lp['msg'])
        (U1, c1), (U2, c2) = map(w, lp['upd'])
        sproj, dproj = _row_call(_int_proj_body, [x],
                                 [W1[:HID], W1[HID:2 * HID], b1], [HID, HID])
        wp2 = _bf(W1[2 * HID:])
        pre = _gather_add(sproj, dproj, pos16, pos16, wp2, src_m, dst_m)
        msg, = _row_call(_edge_body, [pre], [W2, b2], [HID])
        agg = _segment_sum(msg, dst_m, n_pad)
        xn, = _row_call(_upd_int_body, [x, agg, u],
                        [U1[:HID], U1[HID:2 * HID], U1[2 * HID:], c1, U2, c2],
                        [HID])
        return xn

    x = ext_layer(p['ex1'], x)
    for lp in p['internal']:
        x = int_layer(lp, x)
    x = ext_layer(p['ex2'], x)

    (O1, o1), (O2, o2) = map(w, p['out'])
    out, = _row_call(_out_body, [x], [O1, o1, O2, o2], [5])
    return out[:N].reshape(B, Nm, 5)


# speedup vs baseline: 2.0013x; 1.0379x over previous
import functools

import jax
import jax.numpy as jnp
from jax import lax
from jax.experimental import pallas as pl
from jax.experimental.pallas import tpu as pltpu
from jax.experimental.pallas import tpu_sc as plsc

HID = 128
NC, NS, LANES = 2, 16, 16
NW = NC * NS
KB = 128
BN = 1024
FS = HID // 4


def _mm(a, b):
    return jnp.dot(a, b, preferred_element_type=jnp.float32)


def _bf(z):
    return z.astype(jnp.bfloat16).astype(jnp.float32)


def _pos_term(pos, wp):
    return (_bf(pos[:, 0:1]) * _bf(wp[0:1, :])
            + _bf(pos[:, 1:2]) * _bf(wp[1:2, :]))


def _row_call(body, row_ins, full_ins, out_dims):
    nrows = row_ins[0].shape[0]
    grid = (nrows // BN,)
    in_specs = ([pl.BlockSpec((BN, a.shape[1]), lambda i: (i, 0)) for a in row_ins]
                + [pl.BlockSpec(a.shape, lambda i: (0, 0)) for a in full_ins])
    out_shape = [jax.ShapeDtypeStruct((nrows, d), jnp.float32) for d in out_dims]
    out_specs = [pl.BlockSpec((BN, d), lambda i: (i, 0)) for d in out_dims]
    outs = pl.pallas_call(body, grid=grid, in_specs=in_specs,
                          out_specs=out_specs, out_shape=out_shape)(
        *row_ins, *full_ins)
    return outs


def _emb_body(u, pos, wu, wp, b1, w2, b2, o):
    pp = _pos_term(pos, wp)
    h = jnp.tanh(_mm(u[...], wu[...]) + pp + b1[...])
    o[...] = jnp.tanh(_mm(h, w2[...]) + b2[...])


def _int_proj_body(x, ws, wd, b1, s, d):
    s[...] = _mm(x[...], ws[...])
    d[...] = _mm(x[...], wd[...]) + b1[...]


def _ext_dproj_body(x, wx, b1, d):
    d[...] = _mm(x[...], wx[...]) + b1[...]


def _ext_sproj_body(exf, we, s):
    s[...] = _mm(exf[...], we[...])


def _edge_body(pre, dd, wp16, w2, b2, o):
    h = jnp.tanh(pre[...] + _mm(dd[...], wp16[...]))
    o[...] = jnp.tanh(_mm(h, w2[...]) + b2[...])


def _upd_int_body(x, agg, u, ux, ua, uu, c1, u2, c2, o):
    h = jnp.tanh(_mm(x[...], ux[...]) + _mm(agg[...], ua[...])
                 + _mm(u[...], uu[...]) + c1[...])
    o[...] = x[...] + _mm(h, u2[...]) + c2[...]


def _upd_ext_body(x, agg, ux, ua, c1, u2, c2, o):
    h = jnp.tanh(_mm(x[...], ux[...]) + _mm(agg[...], ua[...]) + c1[...])
    o[...] = x[...] + _mm(h, u2[...]) + c2[...]


def _out_body(x, w1, b1, w2, b2, o):
    h = jnp.tanh(_mm(x[...], w1[...]) + b1[...])
    o[...] = _mm(h, w2[...]) + b2[...]


def _sc_mesh():
    return plsc.VectorSubcoreMesh(core_axis_name="c", subcore_axis_name="s",
                                  num_cores=NC, num_subcores=NS)


def _sc_bf(z):
    return lax.convert_element_type(
        lax.convert_element_type(z, jnp.bfloat16), jnp.float32)


def _gather_add(sproj, dproj, pa, pb, src, dst):
    e_pad = src.shape[0]
    ew = e_pad // NW
    nb = ew // KB

    @functools.partial(
        pl.kernel, mesh=_sc_mesh(),
        out_type=[jax.ShapeDtypeStruct((e_pad, HID), jnp.float32),
                  jax.ShapeDtypeStruct((e_pad, 16), jnp.float32)],
        compiler_params=pltpu.CompilerParams(use_tc_tiling_on_sc=False),
        scratch_types=[
            pltpu.VMEM((ew,), jnp.int32),
            pltpu.VMEM((ew,), jnp.int32),
            [pltpu.VMEM((KB, HID), jnp.float32)] * 2,
            [pltpu.VMEM((KB, HID), jnp.float32)] * 2,
            [pltpu.VMEM((KB, 16), jnp.float32)] * 2,
            [pltpu.VMEM((KB, 16), jnp.float32)] * 2,
            [pltpu.SemaphoreType.DMA] * 2,
            [pltpu.SemaphoreType.DMA] * 2,
            [pltpu.SemaphoreType.DMA] * 2,
            [pltpu.SemaphoreType.DMA] * 2,
        ])
    def k(sproj_h, dproj_h, pa_h, pb_h, src_h, dst_h, pre_h, dd_h,
          si, di, av, bv, pav, pbv, sem_a, sem_b, sem_c, sem_d):
        wid = lax.axis_index("s") * NC + lax.axis_index("c")
        base = pl.multiple_of(wid * ew, KB)
        pltpu.sync_copy(src_h.at[pl.ds(base, ew)], si)
        pltpu.sync_copy(dst_h.at[pl.ds(base, ew)], di)

        def issue(i, s):
            lo = pl.multiple_of(i * KB, KB)
            sis = si.at[pl.ds(lo, KB)]
            dis = di.at[pl.ds(lo, KB)]
            pltpu.async_copy(sproj_h.at[sis], av[s], sem_a[s])
            pltpu.async_copy(dproj_h.at[dis], bv[s], sem_b[s])
            pltpu.async_copy(pa_h.at[sis], pav[s], sem_c[s])
            pltpu.async_copy(pb_h.at[dis], pbv[s], sem_d[s])

        issue(0, 0)

        def half(i, s):
            lo = pl.multiple_of(i * KB, KB)
            sis = si.at[pl.ds(lo, KB)]
            dis = di.at[pl.ds(lo, KB)]
            pltpu.make_async_copy(sproj_h.at[sis], av[s], sem_a[s]).wait()
            pltpu.make_async_copy(dproj_h.at[dis], bv[s], sem_b[s]).wait()
            pltpu.make_async_copy(pa_h.at[sis], pav[s], sem_c[s]).wait()
            pltpu.make_async_copy(pb_h.at[dis], pbv[s], sem_d[s]).wait()

            def row(r, _):
                pav[s][r] = _sc_bf(pav[s][r] - pbv[s][r])
                for j in range(HID // LANES):
                    sl = pl.ds(j * LANES, LANES)
                    av[s][r, sl] = av[s][r, sl] + bv[s][r, sl]
                return 0

            lax.fori_loop(0, KB, row, 0)
            off = pl.multiple_of(base + i * KB, KB)
            pltpu.sync_copy(av[s], pre_h.at[pl.ds(off, KB)])
            pltpu.sync_copy(pav[s], dd_h.at[pl.ds(off, KB)])

        def body(g, _):
            i0 = g * 2
            issue(i0 + 1, 1)
            half(i0, 0)

            @pl.when(i0 + 2 < nb)
            def _():
                issue(i0 + 2, 0)

            half(i0 + 1, 1)
            return 0

        lax.fori_loop(0, nb // 2, body, 0)

    return k(sproj, dproj, pa, pb, src, dst)


def _segment_sum(msg, dst, n_pad):
    e_pad = dst.shape[0]
    ew = e_pad // NS
    nb = ew // KB
    zr = n_pad // NS // 8
    wr = n_pad // NS

    @functools.partial(
        pl.kernel, mesh=_sc_mesh(),
        out_type=jax.ShapeDtypeStruct((n_pad, HID), jnp.float32),
        compiler_params=pltpu.CompilerParams(use_tc_tiling_on_sc=False),
        scratch_types=[
            [pltpu.VMEM((KB,), jnp.int32)] * 2,
            [pltpu.VMEM((KB, FS), jnp.float32)] * 2,
            pltpu.VMEM((zr, FS), jnp.float32),
            pltpu.VMEM_SHARED((n_pad, FS), jnp.float32),
            [pltpu.SemaphoreType.DMA] * 2,
            [pltpu.SemaphoreType.DMA] * 2,
        ])
    def k(msg_h, dst_h, agg_h, iv, rows, zb, acc, sem_i, sem_r):
        cid = lax.axis_index("c")
        sid = lax.axis_index("s")

        def zrow(r, _):
            for j in range(FS // LANES):
                zb[r, pl.ds(j * LANES, LANES)] = jnp.zeros((LANES,), jnp.float32)
            return 0

        lax.fori_loop(0, zr, zrow, 0)
        ebase = pl.multiple_of(sid * ew, KB)

        for p in range(2):
            col = (cid * 2 + p) * FS
            for zc in range(8):
                pltpu.sync_copy(zb, acc.at[pl.ds(sid * wr + zc * zr, zr)])
            plsc.subcore_barrier()

            def issue(i, s):
                off = pl.multiple_of(ebase + i * KB, KB)
                pltpu.async_copy(dst_h.at[pl.ds(off, KB)], iv[s], sem_i[s])
                pltpu.async_copy(msg_h.at[pl.ds(off, KB), pl.ds(col, FS)],
                                 rows[s], sem_r[s])

            issue(0, 0)

            def half(i, s):
                off = pl.multiple_of(ebase + i * KB, KB)
                pltpu.make_async_copy(dst_h.at[pl.ds(off, KB)], iv[s],
                                      sem_i[s]).wait()
                pltpu.make_async_copy(
                    msg_h.at[pl.ds(off, KB), pl.ds(col, FS)], rows[s],
                    sem_r[s]).wait()
                pltpu.sync_copy(rows[s], acc.at[iv[s]], add=True)

            def body(g, _):
                i0 = g * 2
                issue(i0 + 1, 1)
                half(i0, 0)

                @pl.when(i0 + 2 < nb)
                def _():
                    issue(i0 + 2, 0)

                half(i0 + 1, 1)
                return 0

            lax.fori_loop(0, nb // 2, body, 0)
            plsc.subcore_barrier()
            pltpu.sync_copy(acc.at[pl.ds(sid * wr, wr)],
                            agg_h.at[pl.ds(sid * wr, wr), pl.ds(col, FS)])
            plsc.subcore_barrier()

    return k(msg, dst)


def _pad_rows(a, n):
    return jnp.pad(a, ((0, n - a.shape[0]), (0, 0)))


def _pad_edges(src, dst, e_pad, pad_node):
    e = src.shape[0]
    src = jnp.pad(src, (0, e_pad - e))
    dst = jnp.pad(dst, (0, e_pad - e), constant_values=pad_node)
    return src, dst


def _round_up(x, m):
    return (x + m - 1) // m * m


def kernel(madis_x, madis_lon, madis_lat, edge_index, ex_lon, ex_lat, ex_x,
           edge_index_e2m, params):
    B, Nm = madis_x.shape[0], madis_x.shape[1]
    Ne = ex_x.shape[1]
    N = B * Nm
    NE = B * Ne
    n_pad = _round_up(N, BN)
    ne_pad = _round_up(NE, BN)

    u = _pad_rows(madis_x.reshape(N, -1), n_pad)
    pos = _pad_rows(jnp.concatenate([madis_lon, madis_lat], axis=2).reshape(N, 2),
                    n_pad)
    exf = _pad_rows(ex_x.reshape(NE, -1), ne_pad)
    ex_pos = _pad_rows(
        jnp.concatenate([ex_lon[..., None], ex_lat[..., None]], axis=2)
        .reshape(NE, 2), ne_pad)

    sh_m = (jnp.arange(B, dtype=jnp.int32) * Nm)[:, None]
    src_m = (edge_index[:, 0, :] + sh_m).reshape(-1)
    dst_m = (edge_index[:, 1, :] + sh_m).reshape(-1)
    sh_e = (jnp.arange(B, dtype=jnp.int32) * Ne)[:, None]
    src_e = (edge_index_e2m[:, 0, :] + sh_e).reshape(-1)
    dst_e = (edge_index_e2m[:, 1, :] + sh_m).reshape(-1)
    em_pad = _round_up(src_m.shape[0], 2 * NW * KB)
    ee_pad = _round_up(src_e.shape[0], 2 * NW * KB)
    src_m, dst_m = _pad_edges(src_m, dst_m, em_pad, N)
    src_e, dst_e = _pad_edges(src_e, dst_e, ee_pad, N)

    pos16 = jnp.pad(pos, ((0, 0), (0, 14)))
    ex_pos16 = jnp.pad(ex_pos, ((0, 0), (0, 14)))

    p = params

    def w(t):
        W, b = t
        return W, b.reshape(1, -1)

    (W1, b1), (W2, b2) = map(w, p['emb'])
    x, = _row_call(_emb_body, [u, pos],
                   [W1[:-2], W1[-2:], b1, W2, b2], [HID])

    def ext_layer(lp, x):
        (W1, b1), (W2, b2) = map(w, lp['msg'])
        (U1, c1), (U2, c2) = map(w, lp['upd'])
        dproj, = _row_call(_ext_dproj_body, [x], [W1[:HID], b1], [HID])
        sproj, = _row_call(_ext_sproj_body, [exf], [W1[HID:HID + 8]], [HID])
        wp16 = jnp.pad(-_bf(W1[HID + 8:]), ((0, 14), (0, 0)))
        pre, dd = _gather_add(sproj, dproj, ex_pos16, pos16, src_e, dst_e)
        msg, = _row_call(_edge_body, [pre, dd], [wp16, W2, b2], [HID])
        agg = _segment_sum(msg, dst_e, n_pad)
        xn, = _row_call(_upd_ext_body, [x, agg],
                        [U1[:HID], U1[HID:], c1, U2, c2], [HID])
        return xn

    def int_layer(lp, x):
        (W1, b1), (W2, b2) = map(w, lp['msg'])
        (U1, c1), (U2, c2) = map(w, lp['upd'])
        sproj, dproj = _row_call(_int_proj_body, [x],
                                 [W1[:HID], W1[HID:2 * HID], b1], [HID, HID])
        wp16 = jnp.pad(_bf(W1[2 * HID:]), ((0, 14), (0, 0)))
        pre, dd = _gather_add(sproj, dproj, pos16, pos16, src_m, dst_m)
        msg, = _row_call(_edge_body, [pre, dd], [wp16, W2, b2], [HID])
        agg = _segment_sum(msg, dst_m, n_pad)
        xn, = _row_call(_upd_int_body, [x, agg, u],
                        [U1[:HID], U1[HID:2 * HID], U1[2 * HID:], c1, U2, c2],
                        [HID])
        return xn

    x = ext_layer(p['ex1'], x)
    for lp in p['internal']:
        x = int_layer(lp, x)
    x = ext_layer(p['ex2'], x)

    (O1, o1), (O2, o2) = map(w, p['out'])
    out, = _row_call(_out_body, [x], [O1, o1, O2, o2], [5])
    return out[:N].reshape(B, Nm, 5)


# speedup vs baseline: 2.0541x; 1.0264x over previous
import functools

import jax
import jax.numpy as jnp
from jax import lax
from jax.experimental import pallas as pl
from jax.experimental.pallas import tpu as pltpu
from jax.experimental.pallas import tpu_sc as plsc

HID = 128
NC, NS, LANES = 2, 16, 16
NW = NC * NS
KB = 128
BN = 1024
FS = HID // 4


def _mm(a, b):
    return jnp.dot(a, b, preferred_element_type=jnp.float32)


def _bf(z):
    return z.astype(jnp.bfloat16).astype(jnp.float32)


def _pos_term(pos, wp):
    return (_bf(pos[:, 0:1]) * _bf(wp[0:1, :])
            + _bf(pos[:, 1:2]) * _bf(wp[1:2, :]))


def _row_call(body, row_ins, full_ins, out_dims):
    nrows = row_ins[0].shape[0]
    grid = (nrows // BN,)
    in_specs = ([pl.BlockSpec((BN, a.shape[1]), lambda i: (i, 0)) for a in row_ins]
                + [pl.BlockSpec(a.shape, lambda i: (0, 0)) for a in full_ins])
    out_shape = [jax.ShapeDtypeStruct((nrows, d), jnp.float32) for d in out_dims]
    out_specs = [pl.BlockSpec((BN, d), lambda i: (i, 0)) for d in out_dims]
    outs = pl.pallas_call(body, grid=grid, in_specs=in_specs,
                          out_specs=out_specs, out_shape=out_shape)(
        *row_ins, *full_ins)
    return outs


def _emb_exdproj_body(u, pos, wu, wp, b1, w2, b2, wx, bx, o, d):
    pp = _pos_term(pos, wp)
    h = jnp.tanh(_mm(u[...], wu[...]) + pp + b1[...])
    x = jnp.tanh(_mm(h, w2[...]) + b2[...])
    o[...] = x
    d[...] = _mm(x, wx[...]) + bx[...]


def _ext_sprojs_body(exf, we_a, we_b, s1, s2):
    s1[...] = _mm(exf[...], we_a[...])
    s2[...] = _mm(exf[...], we_b[...])


def _edge_body(pre, dd, wp16, w2, b2, o):
    h = jnp.tanh(pre[...] + _mm(dd[...], wp16[...]))
    o[...] = jnp.tanh(_mm(h, w2[...]) + b2[...])


def _upd_int(x, agg, u, ux, ua, uu, c1, u2, c2):
    h = jnp.tanh(_mm(x[...], ux[...]) + _mm(agg[...], ua[...])
                 + _mm(u[...], uu[...]) + c1[...])
    return x[...] + _mm(h, u2[...]) + c2[...]


def _upd_ext(x, agg, ux, ua, c1, u2, c2):
    h = jnp.tanh(_mm(x[...], ux[...]) + _mm(agg[...], ua[...]) + c1[...])
    return x[...] + _mm(h, u2[...]) + c2[...]


def _upd_ext_intproj_body(x, agg, ux, ua, c1, u2, c2, ws, wd, b1n, o, s, d):
    xn = _upd_ext(x, agg, ux, ua, c1, u2, c2)
    o[...] = xn
    s[...] = _mm(xn, ws[...])
    d[...] = _mm(xn, wd[...]) + b1n[...]


def _upd_int_intproj_body(x, agg, u, ux, ua, uu, c1, u2, c2, ws, wd, b1n,
                          o, s, d):
    xn = _upd_int(x, agg, u, ux, ua, uu, c1, u2, c2)
    o[...] = xn
    s[...] = _mm(xn, ws[...])
    d[...] = _mm(xn, wd[...]) + b1n[...]


def _upd_int_extproj_body(x, agg, u, ux, ua, uu, c1, u2, c2, wx, bx, o, d):
    xn = _upd_int(x, agg, u, ux, ua, uu, c1, u2, c2)
    o[...] = xn
    d[...] = _mm(xn, wx[...]) + bx[...]


def _upd_ext_out_body(x, agg, ux, ua, c1, u2, c2, o1w, o1b, o2w, o2b, o):
    xn = _upd_ext(x, agg, ux, ua, c1, u2, c2)
    h = jnp.tanh(_mm(xn, o1w[...]) + o1b[...])
    o[...] = _mm(h, o2w[...]) + o2b[...]


def _sc_mesh():
    return plsc.VectorSubcoreMesh(core_axis_name="c", subcore_axis_name="s",
                                  num_cores=NC, num_subcores=NS)


def _sc_bf(z):
    return lax.convert_element_type(
        lax.convert_element_type(z, jnp.bfloat16), jnp.float32)


def _gather_add(sproj, dproj, pa, pb, src, dst):
    e_pad = src.shape[0]
    ew = e_pad // NW
    nb = ew // KB

    @functools.partial(
        pl.kernel, mesh=_sc_mesh(),
        out_type=[jax.ShapeDtypeStruct((e_pad, HID), jnp.float32),
                  jax.ShapeDtypeStruct((e_pad, 16), jnp.float32)],
        compiler_params=pltpu.CompilerParams(use_tc_tiling_on_sc=False),
        scratch_types=[
            pltpu.VMEM((ew,), jnp.int32),
            pltpu.VMEM((ew,), jnp.int32),
            [pltpu.VMEM((KB, HID), jnp.float32)] * 2,
            [pltpu.VMEM((KB, HID), jnp.float32)] * 2,
            [pltpu.VMEM((KB, 16), jnp.float32)] * 2,
            [pltpu.VMEM((KB, 16), jnp.float32)] * 2,
            [pltpu.SemaphoreType.DMA] * 2,
            [pltpu.SemaphoreType.DMA] * 2,
            [pltpu.SemaphoreType.DMA] * 2,
            [pltpu.SemaphoreType.DMA] * 2,
        ])
    def k(sproj_h, dproj_h, pa_h, pb_h, src_h, dst_h, pre_h, dd_h,
          si, di, av, bv, pav, pbv, sem_a, sem_b, sem_c, sem_d):
        wid = lax.axis_index("s") * NC + lax.axis_index("c")
        base = pl.multiple_of(wid * ew, KB)
        pltpu.sync_copy(src_h.at[pl.ds(base, ew)], si)
        pltpu.sync_copy(dst_h.at[pl.ds(base, ew)], di)

        def issue(i, s):
            lo = pl.multiple_of(i * KB, KB)
            sis = si.at[pl.ds(lo, KB)]
            dis = di.at[pl.ds(lo, KB)]
            pltpu.async_copy(sproj_h.at[sis], av[s], sem_a[s])
            pltpu.async_copy(dproj_h.at[dis], bv[s], sem_b[s])
            pltpu.async_copy(pa_h.at[sis], pav[s], sem_c[s])
            pltpu.async_copy(pb_h.at[dis], pbv[s], sem_d[s])

        issue(0, 0)

        def half(i, s):
            lo = pl.multiple_of(i * KB, KB)
            sis = si.at[pl.ds(lo, KB)]
            dis = di.at[pl.ds(lo, KB)]
            pltpu.make_async_copy(sproj_h.at[sis], av[s], sem_a[s]).wait()
            pltpu.make_async_copy(dproj_h.at[dis], bv[s], sem_b[s]).wait()
            pltpu.make_async_copy(pa_h.at[sis], pav[s], sem_c[s]).wait()
            pltpu.make_async_copy(pb_h.at[dis], pbv[s], sem_d[s]).wait()

            def row(t, _):
                for q in range(2):
                    r = t * 2 + q
                    pav[s][r] = _sc_bf(pav[s][r] - pbv[s][r])
                    for j in range(HID // LANES):
                        sl = pl.ds(j * LANES, LANES)
                        av[s][r, sl] = av[s][r, sl] + bv[s][r, sl]
                return 0

            lax.fori_loop(0, KB // 2, row, 0)
            off = pl.multiple_of(base + i * KB, KB)
            pltpu.sync_copy(av[s], pre_h.at[pl.ds(off, KB)])
            pltpu.sync_copy(pav[s], dd_h.at[pl.ds(off, KB)])

        def body(g, _):
            i0 = g * 2
            issue(i0 + 1, 1)
            half(i0, 0)

            @pl.when(i0 + 2 < nb)
            def _():
                issue(i0 + 2, 0)

            half(i0 + 1, 1)
            return 0

        lax.fori_loop(0, nb // 2, body, 0)

    return k(sproj, dproj, pa, pb, src, dst)


def _segment_sum(msg, dst, n_pad):
    e_pad = dst.shape[0]
    ew = e_pad // NS
    nb = ew // KB
    zr = n_pad // NS // 8
    wr = n_pad // NS

    @functools.partial(
        pl.kernel, mesh=_sc_mesh(),
        out_type=jax.ShapeDtypeStruct((n_pad, HID), jnp.float32),
        compiler_params=pltpu.CompilerParams(use_tc_tiling_on_sc=False),
        scratch_types=[
            [pltpu.VMEM((KB,), jnp.int32)] * 2,
            [pltpu.VMEM((KB, FS), jnp.float32)] * 2,
            pltpu.VMEM((zr, FS), jnp.float32),
            pltpu.VMEM_SHARED((n_pad, FS), jnp.float32),
            [pltpu.SemaphoreType.DMA] * 2,
            [pltpu.SemaphoreType.DMA] * 2,
        ])
    def k(msg_h, dst_h, agg_h, iv, rows, zb, acc, sem_i, sem_r):
        cid = lax.axis_index("c")
        sid = lax.axis_index("s")

        def zrow(r, _):
            for j in range(FS // LANES):
                zb[r, pl.ds(j * LANES, LANES)] = jnp.zeros((LANES,), jnp.float32)
            return 0

        lax.fori_loop(0, zr, zrow, 0)
        ebase = pl.multiple_of(sid * ew, KB)

        for p in range(2):
            col = (cid * 2 + p) * FS
            for zc in range(8):
                pltpu.sync_copy(zb, acc.at[pl.ds(sid * wr + zc * zr, zr)])
            plsc.subcore_barrier()

            def issue(i, s):
                off = pl.multiple_of(ebase + i * KB, KB)
                pltpu.async_copy(dst_h.at[pl.ds(off, KB)], iv[s], sem_i[s])
                pltpu.async_copy(msg_h.at[pl.ds(off, KB), pl.ds(col, FS)],
                                 rows[s], sem_r[s])

            issue(0, 0)

            def half(i, s):
                off = pl.multiple_of(ebase + i * KB, KB)
                pltpu.make_async_copy(dst_h.at[pl.ds(off, KB)], iv[s],
                                      sem_i[s]).wait()
                pltpu.make_async_copy(
                    msg_h.at[pl.ds(off, KB), pl.ds(col, FS)], rows[s],
                    sem_r[s]).wait()
                pltpu.sync_copy(rows[s], acc.at[iv[s]], add=True)

            def body(g, _):
                i0 = g * 2
                issue(i0 + 1, 1)
                half(i0, 0)

                @pl.when(i0 + 2 < nb)
                def _():
                    issue(i0 + 2, 0)

                half(i0 + 1, 1)
                return 0

            lax.fori_loop(0, nb // 2, body, 0)
            plsc.subcore_barrier()
            pltpu.sync_copy(acc.at[pl.ds(sid * wr, wr)],
                            agg_h.at[pl.ds(sid * wr, wr), pl.ds(col, FS)])
            plsc.subcore_barrier()

    return k(msg, dst)


def _pad_rows(a, n):
    return jnp.pad(a, ((0, n - a.shape[0]), (0, 0)))


def _pad_edges(src, dst, e_pad, pad_node):
    e = src.shape[0]
    src = jnp.pad(src, (0, e_pad - e))
    dst = jnp.pad(dst, (0, e_pad - e), constant_values=pad_node)
    return src, dst


def _round_up(x, m):
    return (x + m - 1) // m * m


def kernel(madis_x, madis_lon, madis_lat, edge_index, ex_lon, ex_lat, ex_x,
           edge_index_e2m, params):
    B, Nm = madis_x.shape[0], madis_x.shape[1]
    Ne = ex_x.shape[1]
    N = B * Nm
    NE = B * Ne
    n_pad = _round_up(N, BN)
    ne_pad = _round_up(NE, BN)

    u = _pad_rows(madis_x.reshape(N, -1), n_pad)
    pos = _pad_rows(jnp.concatenate([madis_lon, madis_lat], axis=2).reshape(N, 2),
                    n_pad)
    exf = _pad_rows(ex_x.reshape(NE, -1), ne_pad)
    ex_pos = _pad_rows(
        jnp.concatenate([ex_lon[..., None], ex_lat[..., None]], axis=2)
        .reshape(NE, 2), ne_pad)

    sh_m = (jnp.arange(B, dtype=jnp.int32) * Nm)[:, None]
    src_m = (edge_index[:, 0, :] + sh_m).reshape(-1)
    dst_m = (edge_index[:, 1, :] + sh_m).reshape(-1)
    sh_e = (jnp.arange(B, dtype=jnp.int32) * Ne)[:, None]
    src_e = (edge_index_e2m[:, 0, :] + sh_e).reshape(-1)
    dst_e = (edge_index_e2m[:, 1, :] + sh_m).reshape(-1)
    em_pad = _round_up(src_m.shape[0], 2 * NW * KB)
    ee_pad = _round_up(src_e.shape[0], 2 * NW * KB)
    src_m, dst_m = _pad_edges(src_m, dst_m, em_pad, N)
    src_e, dst_e = _pad_edges(src_e, dst_e, ee_pad, N)

    pos16 = jnp.pad(pos, ((0, 0), (0, 14)))
    ex_pos16 = jnp.pad(ex_pos, ((0, 0), (0, 14)))

    p = params

    def w(t):
        W, b = t
        return W, b.reshape(1, -1)

    def msg_w(lp):
        return map(w, lp['msg'])

    def upd_w(lp):
        (U1, c1), (U2, c2) = map(w, lp['upd'])
        return U1, c1, U2, c2

    def ext_edges(lp, sproj, dproj):
        (W1, b1), (W2, b2) = msg_w(lp)
        wp16 = jnp.pad(-_bf(W1[HID + 8:]), ((0, 14), (0, 0)))
        pre, dd = _gather_add(sproj, dproj, ex_pos16, pos16, src_e, dst_e)
        msg, = _row_call(_edge_body, [pre, dd], [wp16, W2, b2], [HID])
        return _segment_sum(msg, dst_e, n_pad)

    def int_edges(lp, sproj, dproj):
        (W1, b1), (W2, b2) = msg_w(lp)
        wp16 = jnp.pad(_bf(W1[2 * HID:]), ((0, 14), (0, 0)))
        pre, dd = _gather_add(sproj, dproj, pos16, pos16, src_m, dst_m)
        msg, = _row_call(_edge_body, [pre, dd], [wp16, W2, b2], [HID])
        return _segment_sum(msg, dst_m, n_pad)

    ex1, ex2 = p['ex1'], p['ex2']
    i1, i2, i3 = p['internal']
    (W1e1, b1e1), _ = msg_w(ex1)
    (W1e2, b1e2), _ = msg_w(ex2)
    (W1i1, b1i1), _ = msg_w(i1)
    (W1i2, b1i2), _ = msg_w(i2)
    (W1i3, b1i3), _ = msg_w(i3)

    sproj_e1, sproj_e2 = _row_call(_ext_sprojs_body, [exf],
                                   [W1e1[HID:HID + 8], W1e2[HID:HID + 8]],
                                   [HID, HID])

    (We1, be1), (We2, be2) = map(w, p['emb'])
    x, dproj = _row_call(_emb_exdproj_body, [u, pos],
                         [We1[:-2], We1[-2:], be1, We2, be2,
                          W1e1[:HID], b1e1], [HID, HID])

    agg = ext_edges(ex1, sproj_e1, dproj)
    U1, c1, U2, c2 = upd_w(ex1)
    x, sproj, dproj = _row_call(
        _upd_ext_intproj_body, [x, agg],
        [U1[:HID], U1[HID:], c1, U2, c2,
         W1i1[:HID], W1i1[HID:2 * HID], b1i1], [HID, HID, HID])

    for lp, (W1n, b1n) in [(i1, (W1i2, b1i2)), (i2, (W1i3, b1i3))]:
        agg = int_edges(lp, sproj, dproj)
        U1, c1, U2, c2 = upd_w(lp)
        x, sproj, dproj = _row_call(
            _upd_int_intproj_body, [x, agg, u],
            [U1[:HID], U1[HID:2 * HID], U1[2 * HID:], c1, U2, c2,
             W1n[:HID], W1n[HID:2 * HID], b1n], [HID, HID, HID])

    agg = int_edges(i3, sproj, dproj)
    U1, c1, U2, c2 = upd_w(i3)
    x, dproj = _row_call(
        _upd_int_extproj_body, [x, agg, u],
        [U1[:HID], U1[HID:2 * HID], U1[2 * HID:], c1, U2, c2,
         W1e2[:HID], b1e2], [HID, HID])

    agg = ext_edges(ex2, sproj_e2, dproj)
    U1, c1, U2, c2 = upd_w(ex2)
    (O1, o1), (O2, o2) = map(w, p['out'])
    out, = _row_call(_upd_ext_out_body, [x, agg],
                     [U1[:HID], U1[HID:], c1, U2, c2, O1, o1, O2, o2], [5])
    return out[:N].reshape(B, Nm, 5)


# speedup vs baseline: 2.0549x; 1.0004x over previous
import functools

import jax
import jax.numpy as jnp
from jax import lax
from jax.experimental import pallas as pl
from jax.experimental.pallas import tpu as pltpu
from jax.experimental.pallas import tpu_sc as plsc

HID = 128
NC, NS, LANES = 2, 16, 16
NW = NC * NS
KB = 128
BN = 1024
FS = HID // 4


def _mm(a, b):
    return jnp.dot(a, b, preferred_element_type=jnp.float32)


def _bf(z):
    return z.astype(jnp.bfloat16).astype(jnp.float32)


def _pos_term(pos, wp):
    return (_bf(pos[:, 0:1]) * _bf(wp[0:1, :])
            + _bf(pos[:, 1:2]) * _bf(wp[1:2, :]))


def _row_call(body, row_ins, full_ins, out_dims):
    nrows = row_ins[0].shape[0]
    grid = (nrows // BN,)
    in_specs = ([pl.BlockSpec((BN, a.shape[1]), lambda i: (i, 0)) for a in row_ins]
                + [pl.BlockSpec(a.shape, lambda i: (0, 0)) for a in full_ins])
    out_shape = [jax.ShapeDtypeStruct((nrows, d), jnp.float32) for d in out_dims]
    out_specs = [pl.BlockSpec((BN, d), lambda i: (i, 0)) for d in out_dims]
    outs = pl.pallas_call(body, grid=grid, in_specs=in_specs,
                          out_specs=out_specs, out_shape=out_shape)(
        *row_ins, *full_ins)
    return outs


def _emb_exdproj_body(u, pos, wu, wp, b1, w2, b2, wx, bx, o, d):
    pp = _pos_term(pos, wp)
    h = jnp.tanh(_mm(u[...], wu[...]) + pp + b1[...])
    x = jnp.tanh(_mm(h, w2[...]) + b2[...])
    o[...] = x
    d[...] = _mm(x, wx[...]) + bx[...]


def _ext_sprojs_body(exf, we_a, we_b, s1, s2):
    s1[...] = _mm(exf[...], we_a[...])
    s2[...] = _mm(exf[...], we_b[...])


def _edge_body(pre, dd, wp16, w2, b2, o):
    h = jnp.tanh(pre[...] + _mm(dd[...], wp16[...]))
    o[...] = jnp.tanh(_mm(h, w2[...]) + b2[...])


def _upd_int(x, agg, u, ux, ua, uu, c1, u2, c2):
    h = jnp.tanh(_mm(x[...], ux[...]) + _mm(agg[...], ua[...])
                 + _mm(u[...], uu[...]) + c1[...])
    return x[...] + _mm(h, u2[...]) + c2[...]


def _upd_ext(x, agg, ux, ua, c1, u2, c2):
    h = jnp.tanh(_mm(x[...], ux[...]) + _mm(agg[...], ua[...]) + c1[...])
    return x[...] + _mm(h, u2[...]) + c2[...]


def _upd_ext_intproj_body(x, agg, ux, ua, c1, u2, c2, ws, wd, b1n, o, s, d):
    xn = _upd_ext(x, agg, ux, ua, c1, u2, c2)
    o[...] = xn
    s[...] = _mm(xn, ws[...])
    d[...] = _mm(xn, wd[...]) + b1n[...]


def _upd_int_intproj_body(x, agg, u, ux, ua, uu, c1, u2, c2, ws, wd, b1n,
                          o, s, d):
    xn = _upd_int(x, agg, u, ux, ua, uu, c1, u2, c2)
    o[...] = xn
    s[...] = _mm(xn, ws[...])
    d[...] = _mm(xn, wd[...]) + b1n[...]


def _upd_int_extproj_body(x, agg, u, ux, ua, uu, c1, u2, c2, wx, bx, o, d):
    xn = _upd_int(x, agg, u, ux, ua, uu, c1, u2, c2)
    o[...] = xn
    d[...] = _mm(xn, wx[...]) + bx[...]


def _upd_ext_out_body(x, agg, ux, ua, c1, u2, c2, o1w, o1b, o2w, o2b, o):
    xn = _upd_ext(x, agg, ux, ua, c1, u2, c2)
    h = jnp.tanh(_mm(xn, o1w[...]) + o1b[...])
    o[...] = _mm(h, o2w[...]) + o2b[...]


def _sc_mesh():
    return plsc.VectorSubcoreMesh(core_axis_name="c", subcore_axis_name="s",
                                  num_cores=NC, num_subcores=NS)


def _sc_bf(z):
    return lax.convert_element_type(
        lax.convert_element_type(z, jnp.bfloat16), jnp.float32)


def _gather_add(sproj, dproj, pa, pb, src, dst):
    e_pad = src.shape[0]
    ew = e_pad // NW
    nb = ew // KB

    @functools.partial(
        pl.kernel, mesh=_sc_mesh(),
        out_type=[jax.ShapeDtypeStruct((e_pad, HID), jnp.float32),
                  jax.ShapeDtypeStruct((e_pad, 16), jnp.float32)],
        compiler_params=pltpu.CompilerParams(use_tc_tiling_on_sc=False),
        scratch_types=[
            pltpu.VMEM((ew,), jnp.int32),
            pltpu.VMEM((ew,), jnp.int32),
            [pltpu.VMEM((KB, HID), jnp.float32)] * 2,
            [pltpu.VMEM((KB, HID), jnp.float32)] * 2,
            [pltpu.VMEM((KB, 16), jnp.float32)] * 2,
            [pltpu.VMEM((KB, 16), jnp.float32)] * 2,
            [pltpu.SemaphoreType.DMA] * 2,
            [pltpu.SemaphoreType.DMA] * 2,
            [pltpu.SemaphoreType.DMA] * 2,
            [pltpu.SemaphoreType.DMA] * 2,
        ])
    def k(sproj_h, dproj_h, pa_h, pb_h, src_h, dst_h, pre_h, dd_h,
          si, di, av, bv, pav, pbv, sem_a, sem_b, sem_c, sem_d):
        wid = lax.axis_index("s") * NC + lax.axis_index("c")
        base = pl.multiple_of(wid * ew, KB)
        pltpu.sync_copy(src_h.at[pl.ds(base, ew)], si)
        pltpu.sync_copy(dst_h.at[pl.ds(base, ew)], di)

        def issue(i, s):
            lo = pl.multiple_of(i * KB, KB)
            sis = si.at[pl.ds(lo, KB)]
            dis = di.at[pl.ds(lo, KB)]
            pltpu.async_copy(sproj_h.at[sis], av[s], sem_a[s])
            pltpu.async_copy(dproj_h.at[dis], bv[s], sem_b[s])
            pltpu.async_copy(pa_h.at[sis], pav[s], sem_c[s])
            pltpu.async_copy(pb_h.at[dis], pbv[s], sem_d[s])

        issue(0, 0)

        def half(i, s):
            lo = pl.multiple_of(i * KB, KB)
            sis = si.at[pl.ds(lo, KB)]
            dis = di.at[pl.ds(lo, KB)]
            pltpu.make_async_copy(sproj_h.at[sis], av[s], sem_a[s]).wait()
            pltpu.make_async_copy(dproj_h.at[dis], bv[s], sem_b[s]).wait()
            pltpu.make_async_copy(pa_h.at[sis], pav[s], sem_c[s]).wait()
            pltpu.make_async_copy(pb_h.at[dis], pbv[s], sem_d[s]).wait()

            def row(t, _):
                for q in range(4):
                    r = t * 4 + q
                    pav[s][r] = _sc_bf(pav[s][r] - pbv[s][r])
                    for j in range(HID // LANES):
                        sl = pl.ds(j * LANES, LANES)
                        av[s][r, sl] = av[s][r, sl] + bv[s][r, sl]
                return 0

            lax.fori_loop(0, KB // 4, row, 0)
            off = pl.multiple_of(base + i * KB, KB)
            pltpu.sync_copy(av[s], pre_h.at[pl.ds(off, KB)])
            pltpu.sync_copy(pav[s], dd_h.at[pl.ds(off, KB)])

        def body(g, _):
            i0 = g * 2
            issue(i0 + 1, 1)
            half(i0, 0)

            @pl.when(i0 + 2 < nb)
            def _():
                issue(i0 + 2, 0)

            half(i0 + 1, 1)
            return 0

        lax.fori_loop(0, nb // 2, body, 0)

    return k(sproj, dproj, pa, pb, src, dst)


def _segment_sum(msg, dst, n_pad):
    e_pad = dst.shape[0]
    ew = e_pad // NS
    nb = ew // KB
    zr = n_pad // NS // 8
    wr = n_pad // NS

    @functools.partial(
        pl.kernel, mesh=_sc_mesh(),
        out_type=jax.ShapeDtypeStruct((n_pad, HID), jnp.float32),
        compiler_params=pltpu.CompilerParams(use_tc_tiling_on_sc=False),
        scratch_types=[
            [pltpu.VMEM((KB,), jnp.int32)] * 2,
            [pltpu.VMEM((KB, FS), jnp.float32)] * 2,
            pltpu.VMEM((zr, FS), jnp.float32),
            pltpu.VMEM_SHARED((n_pad, FS), jnp.float32),
            [pltpu.SemaphoreType.DMA] * 2,
            [pltpu.SemaphoreType.DMA] * 2,
        ])
    def k(msg_h, dst_h, agg_h, iv, rows, zb, acc, sem_i, sem_r):
        cid = lax.axis_index("c")
        sid = lax.axis_index("s")

        def zrow(r, _):
            for j in range(FS // LANES):
                zb[r, pl.ds(j * LANES, LANES)] = jnp.zeros((LANES,), jnp.float32)
            return 0

        lax.fori_loop(0, zr, zrow, 0)
        ebase = pl.multiple_of(sid * ew, KB)

        for p in range(2):
            col = (cid * 2 + p) * FS
            for zc in range(8):
                pltpu.sync_copy(zb, acc.at[pl.ds(sid * wr + zc * zr, zr)])
            plsc.subcore_barrier()

            def issue(i, s):
                off = pl.multiple_of(ebase + i * KB, KB)
                pltpu.async_copy(dst_h.at[pl.ds(off, KB)], iv[s], sem_i[s])
                pltpu.async_copy(msg_h.at[pl.ds(off, KB), pl.ds(col, FS)],
                                 rows[s], sem_r[s])

            issue(0, 0)

            def half(i, s):
                off = pl.multiple_of(ebase + i * KB, KB)
                pltpu.make_async_copy(dst_h.at[pl.ds(off, KB)], iv[s],
                                      sem_i[s]).wait()
                pltpu.make_async_copy(
                    msg_h.at[pl.ds(off, KB), pl.ds(col, FS)], rows[s],
                    sem_r[s]).wait()
                pltpu.sync_copy(rows[s], acc.at[iv[s]], add=True)

            def body(g, _):
                i0 = g * 2
                issue(i0 + 1, 1)
                half(i0, 0)

                @pl.when(i0 + 2 < nb)
                def _():
                    issue(i0 + 2, 0)

                half(i0 + 1, 1)
                return 0

            lax.fori_loop(0, nb // 2, body, 0)
            plsc.subcore_barrier()
            pltpu.sync_copy(acc.at[pl.ds(sid * wr, wr)],
                            agg_h.at[pl.ds(sid * wr, wr), pl.ds(col, FS)])
            plsc.subcore_barrier()

    return k(msg, dst)


def _pad_rows(a, n):
    return jnp.pad(a, ((0, n - a.shape[0]), (0, 0)))


def _pad_edges(src, dst, e_pad, pad_node):
    e = src.shape[0]
    src = jnp.pad(src, (0, e_pad - e))
    dst = jnp.pad(dst, (0, e_pad - e), constant_values=pad_node)
    return src, dst


def _round_up(x, m):
    return (x + m - 1) // m * m


def kernel(madis_x, madis_lon, madis_lat, edge_index, ex_lon, ex_lat, ex_x,
           edge_index_e2m, params):
    B, Nm = madis_x.shape[0], madis_x.shape[1]
    Ne = ex_x.shape[1]
    N = B * Nm
    NE = B * Ne
    n_pad = _round_up(N, BN)
    ne_pad = _round_up(NE, BN)

    u = _pad_rows(madis_x.reshape(N, -1), n_pad)
    pos = _pad_rows(jnp.concatenate([madis_lon, madis_lat], axis=2).reshape(N, 2),
                    n_pad)
    exf = _pad_rows(ex_x.reshape(NE, -1), ne_pad)
    ex_pos = _pad_rows(
        jnp.concatenate([ex_lon[..., None], ex_lat[..., None]], axis=2)
        .reshape(NE, 2), ne_pad)

    sh_m = (jnp.arange(B, dtype=jnp.int32) * Nm)[:, None]
    src_m = (edge_index[:, 0, :] + sh_m).reshape(-1)
    dst_m = (edge_index[:, 1, :] + sh_m).reshape(-1)
    sh_e = (jnp.arange(B, dtype=jnp.int32) * Ne)[:, None]
    src_e = (edge_index_e2m[:, 0, :] + sh_e).reshape(-1)
    dst_e = (edge_index_e2m[:, 1, :] + sh_m).reshape(-1)
    em_pad = _round_up(src_m.shape[0], 2 * NW * KB)
    ee_pad = _round_up(src_e.shape[0], 2 * NW * KB)
    src_m, dst_m = _pad_edges(src_m, dst_m, em_pad, N)
    src_e, dst_e = _pad_edges(src_e, dst_e, ee_pad, N)

    pos16 = jnp.pad(pos, ((0, 0), (0, 14)))
    ex_pos16 = jnp.pad(ex_pos, ((0, 0), (0, 14)))

    p = params

    def w(t):
        W, b = t
        return W, b.reshape(1, -1)

    def msg_w(lp):
        return map(w, lp['msg'])

    def upd_w(lp):
        (U1, c1), (U2, c2) = map(w, lp['upd'])
        return U1, c1, U2, c2

    def ext_edges(lp, sproj, dproj):
        (W1, b1), (W2, b2) = msg_w(lp)
        wp16 = jnp.pad(-_bf(W1[HID + 8:]), ((0, 14), (0, 0)))
        pre, dd = _gather_add(sproj, dproj, ex_pos16, pos16, src_e, dst_e)
        msg, = _row_call(_edge_body, [pre, dd], [wp16, W2, b2], [HID])
        return _segment_sum(msg, dst_e, n_pad)

    def int_edges(lp, sproj, dproj):
        (W1, b1), (W2, b2) = msg_w(lp)
        wp16 = jnp.pad(_bf(W1[2 * HID:]), ((0, 14), (0, 0)))
        pre, dd = _gather_add(sproj, dproj, pos16, pos16, src_m, dst_m)
        msg, = _row_call(_edge_body, [pre, dd], [wp16, W2, b2], [HID])
        return _segment_sum(msg, dst_m, n_pad)

    ex1, ex2 = p['ex1'], p['ex2']
    i1, i2, i3 = p['internal']
    (W1e1, b1e1), _ = msg_w(ex1)
    (W1e2, b1e2), _ = msg_w(ex2)
    (W1i1, b1i1), _ = msg_w(i1)
    (W1i2, b1i2), _ = msg_w(i2)
    (W1i3, b1i3), _ = msg_w(i3)

    sproj_e1, sproj_e2 = _row_call(_ext_sprojs_body, [exf],
                                   [W1e1[HID:HID + 8], W1e2[HID:HID + 8]],
                                   [HID, HID])

    (We1, be1), (We2, be2) = map(w, p['emb'])
    x, dproj = _row_call(_emb_exdproj_body, [u, pos],
                         [We1[:-2], We1[-2:], be1, We2, be2,
                          W1e1[:HID], b1e1], [HID, HID])

    agg = ext_edges(ex1, sproj_e1, dproj)
    U1, c1, U2, c2 = upd_w(ex1)
    x, sproj, dproj = _row_call(
        _upd_ext_intproj_body, [x, agg],
        [U1[:HID], U1[HID:], c1, U2, c2,
         W1i1[:HID], W1i1[HID:2 * HID], b1i1], [HID, HID, HID])

    for lp, (W1n, b1n) in [(i1, (W1i2, b1i2)), (i2, (W1i3, b1i3))]:
        agg = int_edges(lp, sproj, dproj)
        U1, c1, U2, c2 = upd_w(lp)
        x, sproj, dproj = _row_call(
            _upd_int_intproj_body, [x, agg, u],
            [U1[:HID], U1[HID:2 * HID], U1[2 * HID:], c1, U2, c2,
             W1n[:HID], W1n[HID:2 * HID], b1n], [HID, HID, HID])

    agg = int_edges(i3, sproj, dproj)
    U1, c1, U2, c2 = upd_w(i3)
    x, dproj = _row_call(
        _upd_int_extproj_body, [x, agg, u],
        [U1[:HID], U1[HID:2 * HID], U1[2 * HID:], c1, U2, c2,
         W1e2[:HID], b1e2], [HID, HID])

    agg = ext_edges(ex2, sproj_e2, dproj)
    U1, c1, U2, c2 = upd_w(ex2)
    (O1, o1), (O2, o2) = map(w, p['out'])
    out, = _row_call(_upd_ext_out_body, [x, agg],
                     [U1[:HID], U1[HID:], c1, U2, c2, O1, o1, O2, o2], [5])
    return out[:N].reshape(B, Nm, 5)
